# baseline (speedup 1.0000x reference)
import jax
import jax.numpy as jnp
from jax import lax
from jax.experimental import pallas as pl
from jax.experimental.pallas import tpu as pltpu

N = 1024
D = 64
K = 30
ALPHA = 3.0
PROP_ALPHA = 0.05
C1 = 32
T = 48


def _leaky(v):
    return jnp.where(v >= 0, v, 0.01 * v)


def _bf(v):
    return v.astype(jnp.bfloat16)


def _dot(a, b, dims):
    return lax.dot_general(_bf(a), _bf(b), (dims, ((), ())),
                           preferred_element_type=jnp.float32)


def _graph_kernel(emb1_ref, emb2_ref, l1w_ref, l1b_ref, l2w_ref, l2b_ref,
                  anormt_ref, work_ref, mask_ref):
    f32 = jnp.float32
    n1 = jnp.tanh(ALPHA * (_dot(emb1_ref[...], l1w_ref[...], ((1,), (1,)))
                           + l1b_ref[...][None, :]))
    n2 = jnp.tanh(ALPHA * (_dot(emb2_ref[...], l2w_ref[...], ((1,), (1,)))
                           + l2b_ref[...][None, :]))
    s1 = _dot(n1, n2, ((1,), (1,)))
    s2 = _dot(n2, n1, ((1,), (1,)))
    b = jax.nn.relu(jnp.tanh(ALPHA * (s2 - s1)))

    rowidx = lax.broadcasted_iota(jnp.int32, (N, N), 0)

    work_ref[...] = b
    mask_ref[...] = jnp.zeros((N, N), f32)

    def body(_, carry):
        work = work_ref[...]
        m = jnp.max(work, axis=0, keepdims=True)
        is_max = work == m
        sel_row = jnp.min(jnp.where(is_max, rowidx, N), axis=0, keepdims=True)
        sel = rowidx == sel_row
        mask_ref[...] = mask_ref[...] + sel.astype(f32)
        work_ref[...] = jnp.where(sel, -1.0, work)
        return carry

    lax.fori_loop(0, K, body, 0)
    adjt = jnp.where(mask_ref[...] > 0, b, 0.0)
    att = adjt + (rowidx == lax.broadcasted_iota(jnp.int32, (N, N), 1)
                  ).astype(f32)
    sums = jnp.sum(att, axis=0, keepdims=True)
    anormt_ref[...] = att / sums


def _main_kernel(x_ref, anormt_ref, w_in_ref, b_in_ref, mlp_w_ref, mlp_b_ref,
                 gen_w_ref, gen_b_ref, out_ref, planes_ref):
    f32 = jnp.float32
    xb = x_ref[0]
    w_in = _bf(w_in_ref[...].reshape(1, C1, 1)).astype(f32)
    b_in = b_in_ref[...].reshape(1, C1, 1)
    xb16 = _bf(xb).astype(f32)
    h0 = _leaky(xb16[:, None, :] * w_in + b_in)
    h0f = h0.reshape(T * C1, N)
    prop = _dot(h0f, anormt_ref[...], ((1,), (0,)))
    h1f = PROP_ALPHA * h0f + (1.0 - PROP_ALPHA) * prop

    mw = mlp_w_ref[...].reshape(C1, 2 * C1)
    mwa, mwb = mw[:, :C1], mw[:, C1:]
    mlp_b = mlp_b_ref[...][:, None]
    gw = gen_w_ref[...].reshape(C1, 9)

    for t in range(T):
        h0_t = h0f[t * C1:(t + 1) * C1, :]
        h1_t = h1f[t * C1:(t + 1) * C1, :]
        lat_t = (_dot(mwa, h0_t, ((1,), (0,)))
                 + _dot(mwb, h1_t, ((1,), (0,)))
                 + mlp_b)
        planes_ref[:, t, :] = _dot(gw, lat_t, ((0,), (0,)))

    planes = planes_ref[...]
    tix = lax.broadcasted_iota(jnp.int32, (T, N), 0)
    nix = lax.broadcasted_iota(jnp.int32, (T, N), 1)
    acc = jnp.zeros((T, N), f32)
    for kh in range(3):
        for kw in range(3):
            di, dj = kh - 1, kw - 1
            shifted = planes[kh * 3 + kw]
            if dj:
                shifted = jnp.roll(shifted, -dj, axis=0)
            if di:
                shifted = jnp.roll(shifted, -di, axis=1)
            valid = ((tix + dj >= 0) & (tix + dj < T)
                     & (nix + di >= 0) & (nix + di < N))
            acc = acc + jnp.where(valid, shifted, 0.0)
    out_ref[0] = _leaky(acc + gen_b_ref[...])


def kernel(x, idx, device, emb1, emb2, lin1_w, lin1_b, lin2_w, lin2_b,
           conv_in_w, conv_in_b, mlp_w, mlp_b, gen_w, gen_b):
    del idx, device
    f32 = jnp.float32

    anormt = pl.pallas_call(
        _graph_kernel,
        out_shape=jax.ShapeDtypeStruct((N, N), f32),
        scratch_shapes=[pltpu.VMEM((N, N), f32), pltpu.VMEM((N, N), f32)],
    )(emb1, emb2, lin1_w, lin1_b, lin2_w, lin2_b)

    out = pl.pallas_call(
        _main_kernel,
        grid=(x.shape[0],),
        in_specs=[
            pl.BlockSpec((1, T, N), lambda b: (b, 0, 0)),
            pl.BlockSpec((N, N), lambda b: (0, 0)),
            pl.BlockSpec(conv_in_w.shape, lambda b: (0,) * 4),
            pl.BlockSpec(conv_in_b.shape, lambda b: (0,)),
            pl.BlockSpec(mlp_w.shape, lambda b: (0,) * 4),
            pl.BlockSpec(mlp_b.shape, lambda b: (0,)),
            pl.BlockSpec(gen_w.shape, lambda b: (0,) * 4),
            pl.BlockSpec(gen_b.shape, lambda b: (0,)),
        ],
        out_specs=pl.BlockSpec((1, T, N), lambda b: (b, 0, 0)),
        out_shape=jax.ShapeDtypeStruct((x.shape[0], T, N), f32),
        scratch_shapes=[pltpu.VMEM((9, T, N), f32)],
    )(x, anormt, conv_in_w, conv_in_b, mlp_w, mlp_b, gen_w, gen_b)
    return out


# speedup vs baseline: 2.1792x; 2.1792x over previous
import jax
import jax.numpy as jnp
from jax import lax
from jax.experimental import pallas as pl
from jax.experimental.pallas import tpu as pltpu

N = 1024
D = 64
K = 30
ALPHA = 3.0
PROP_ALPHA = 0.05
C1 = 32
T = 48
TG = 8
NCHUNK = T // TG


def _leaky(v):
    return jnp.where(v >= 0, v, 0.01 * v)


def _bf(v):
    return v.astype(jnp.bfloat16)


def _dot16(a16, b16, dims):
    return lax.dot_general(a16, b16, (dims, ((), ())),
                           preferred_element_type=jnp.float32)


def _dot(a, b, dims):
    return _dot16(_bf(a), _bf(b), dims)


def _graph_kernel(emb1_ref, emb2_ref, l1w_ref, l1b_ref, l2w_ref, l2b_ref,
                  anormt_ref, work_ref, mask_ref):
    f32 = jnp.float32
    n1 = jnp.tanh(ALPHA * (_dot(emb1_ref[...], l1w_ref[...], ((1,), (1,)))
                           + l1b_ref[...][None, :]))
    n2 = jnp.tanh(ALPHA * (_dot(emb2_ref[...], l2w_ref[...], ((1,), (1,)))
                           + l2b_ref[...][None, :]))
    s1 = _dot(n1, n2, ((1,), (1,)))
    s2 = _dot(n2, n1, ((1,), (1,)))
    b = jax.nn.relu(jnp.tanh(ALPHA * (s2 - s1)))

    rowidx = lax.broadcasted_iota(jnp.int32, (N, N), 0)

    ones = b == 1.0
    cnt1 = jnp.sum(ones.astype(jnp.int32), axis=0)
    all_saturated = jnp.min(cnt1) >= K

    @pl.when(all_saturated)
    def _fast():
        ltri = (rowidx >= lax.broadcasted_iota(jnp.int32, (N, N), 1)
                ).astype(jnp.bfloat16)
        csum = _dot16(ltri, ones.astype(jnp.bfloat16), ((1,), (0,)))
        mask_ref[...] = (ones & (csum <= K)).astype(f32)

    @pl.when(jnp.logical_not(all_saturated))
    def _general():
        work_ref[...] = b
        mask_ref[...] = jnp.zeros((N, N), f32)

        def body(_, carry):
            work = work_ref[...]
            m = jnp.max(work, axis=0, keepdims=True)
            is_max = work == m
            sel_row = jnp.min(jnp.where(is_max, rowidx, N), axis=0,
                              keepdims=True)
            sel = rowidx == sel_row
            mask_ref[...] = mask_ref[...] + sel.astype(f32)
            work_ref[...] = jnp.where(sel, -1.0, work)
            return carry

        lax.fori_loop(0, K, body, 0)

    adjt = jnp.where(mask_ref[...] > 0, b, 0.0)
    att = adjt + (rowidx == lax.broadcasted_iota(jnp.int32, (N, N), 1)
                  ).astype(f32)
    sums = jnp.sum(att, axis=0, keepdims=True)
    anormt_ref[...] = att / sums


def _main_kernel(x_ref, anormt_ref, w_in_ref, b_in_ref, wa8_ref, wb8_ref,
                 wg8_ref, mlpb8_ref, gen_b_ref, out_ref, planes_ref):
    f32 = jnp.float32
    xb = x_ref[0]
    w_in = _bf(w_in_ref[...].reshape(1, C1, 1)).astype(f32)
    b_in = b_in_ref[...].reshape(1, C1, 1)
    xb16 = _bf(xb).astype(f32)
    h0 = _leaky(xb16[:, None, :] * w_in + b_in)
    h0f = h0.reshape(T * C1, N)
    h0f16 = _bf(h0f)
    prop = _dot16(h0f16, _bf(anormt_ref[...]), ((1,), (0,)))
    h1f16 = _bf(PROP_ALPHA * h0f + (1.0 - PROP_ALPHA) * prop)

    wa8 = wa8_ref[...]
    wb8 = wb8_ref[...]
    wg8 = wg8_ref[...]
    mlpb8 = mlpb8_ref[...][:, None]

    for g in range(NCHUNK):
        rows = slice(g * TG * C1, (g + 1) * TG * C1)
        lat = (_dot16(wa8, h0f16[rows], ((1,), (0,)))
               + _dot16(wb8, h1f16[rows], ((1,), (0,)))
               + mlpb8)
        pchunk = _dot16(wg8, _bf(lat), ((1,), (0,)))
        for p in range(9):
            planes_ref[p, g * TG:(g + 1) * TG, :] = (
                pchunk[p * TG:(p + 1) * TG])

    planes = planes_ref[...]
    tix = lax.broadcasted_iota(jnp.int32, (T, N), 0)
    nix = lax.broadcasted_iota(jnp.int32, (T, N), 1)
    acc = jnp.zeros((T, N), f32)
    for kh in range(3):
        for kw in range(3):
            di, dj = kh - 1, kw - 1
            shifted = planes[kh * 3 + kw]
            if dj:
                shifted = jnp.roll(shifted, -dj, axis=0)
            if di:
                shifted = jnp.roll(shifted, -di, axis=1)
            valid = ((tix + dj >= 0) & (tix + dj < T)
                     & (nix + di >= 0) & (nix + di < N))
            acc = acc + jnp.where(valid, shifted, 0.0)
    out_ref[0] = _leaky(acc + gen_b_ref[...])


def kernel(x, idx, device, emb1, emb2, lin1_w, lin1_b, lin2_w, lin2_b,
           conv_in_w, conv_in_b, mlp_w, mlp_b, gen_w, gen_b):
    del idx, device
    f32 = jnp.float32

    anormt = pl.pallas_call(
        _graph_kernel,
        out_shape=jax.ShapeDtypeStruct((N, N), f32),
        scratch_shapes=[pltpu.VMEM((N, N), f32), pltpu.VMEM((N, N), f32)],
    )(emb1, emb2, lin1_w, lin1_b, lin2_w, lin2_b)

    eye8 = jnp.eye(TG, dtype=jnp.bfloat16)
    mw = _bf(mlp_w.reshape(C1, 2 * C1))
    wa8 = jnp.kron(eye8, mw[:, :C1])
    wb8 = jnp.kron(eye8, mw[:, C1:])
    gwt = _bf(gen_w.reshape(C1, 9)).T
    perm = jnp.arange(9 * TG).reshape(9, TG)
    perm = (perm % TG) * 9 + perm // TG
    wg8 = jnp.kron(eye8, gwt)[perm.reshape(-1)]
    mlpb8 = jnp.tile(mlp_b, TG)

    out = pl.pallas_call(
        _main_kernel,
        grid=(x.shape[0],),
        in_specs=[
            pl.BlockSpec((1, T, N), lambda b: (b, 0, 0)),
            pl.BlockSpec((N, N), lambda b: (0, 0)),
            pl.BlockSpec(conv_in_w.shape, lambda b: (0,) * 4),
            pl.BlockSpec(conv_in_b.shape, lambda b: (0,)),
            pl.BlockSpec(wa8.shape, lambda b: (0, 0)),
            pl.BlockSpec(wb8.shape, lambda b: (0, 0)),
            pl.BlockSpec(wg8.shape, lambda b: (0, 0)),
            pl.BlockSpec(mlpb8.shape, lambda b: (0,)),
            pl.BlockSpec(gen_b.shape, lambda b: (0,)),
        ],
        out_specs=pl.BlockSpec((1, T, N), lambda b: (b, 0, 0)),
        out_shape=jax.ShapeDtypeStruct((x.shape[0], T, N), f32),
        scratch_shapes=[pltpu.VMEM((9, T, N), f32)],
    )(x, anormt, conv_in_w, conv_in_b, wa8, wb8, wg8, mlpb8, gen_b)
    return out


# speedup vs baseline: 2.2066x; 1.0126x over previous
import jax
import jax.numpy as jnp
from jax import lax
from jax.experimental import pallas as pl
from jax.experimental.pallas import tpu as pltpu

N = 1024
D = 64
K = 30
ALPHA = 3.0
PROP_ALPHA = 0.05
C1 = 32
T = 48
TG = 8
NCHUNK = T // TG


def _leaky(v):
    return jnp.where(v >= 0, v, 0.01 * v)


def _bf(v):
    return v.astype(jnp.bfloat16)


def _dot16(a16, b16, dims):
    return lax.dot_general(a16, b16, (dims, ((), ())),
                           preferred_element_type=jnp.float32)


def _dot(a, b, dims):
    return _dot16(_bf(a), _bf(b), dims)


def _graph_kernel(emb1_ref, emb2_ref, l1w_ref, l1b_ref, l2w_ref, l2b_ref,
                  anormt_ref, work_ref, mask_ref):
    f32 = jnp.float32
    n1 = jnp.tanh(ALPHA * (_dot(emb1_ref[...], l1w_ref[...], ((1,), (1,)))
                           + l1b_ref[...][None, :]))
    n2 = jnp.tanh(ALPHA * (_dot(emb2_ref[...], l2w_ref[...], ((1,), (1,)))
                           + l2b_ref[...][None, :]))
    s1 = _dot(n1, n2, ((1,), (1,)))
    s2 = _dot(n2, n1, ((1,), (1,)))
    b = jax.nn.relu(jnp.tanh(ALPHA * (s2 - s1)))

    rowidx = lax.broadcasted_iota(jnp.int32, (N, N), 0)

    ones = b == 1.0
    cnt1 = jnp.sum(ones.astype(jnp.int32), axis=0)
    all_saturated = jnp.min(cnt1) >= K

    @pl.when(all_saturated)
    def _fast():
        ltri = (rowidx >= lax.broadcasted_iota(jnp.int32, (N, N), 1)
                ).astype(jnp.bfloat16)
        csum = _dot16(ltri, ones.astype(jnp.bfloat16), ((1,), (0,)))
        mask_ref[...] = (ones & (csum <= K)).astype(f32)

    @pl.when(jnp.logical_not(all_saturated))
    def _general():
        work_ref[...] = b
        mask_ref[...] = jnp.zeros((N, N), f32)

        def body(_, carry):
            work = work_ref[...]
            m = jnp.max(work, axis=0, keepdims=True)
            is_max = work == m
            sel_row = jnp.min(jnp.where(is_max, rowidx, N), axis=0,
                              keepdims=True)
            sel = rowidx == sel_row
            mask_ref[...] = mask_ref[...] + sel.astype(f32)
            work_ref[...] = jnp.where(sel, -1.0, work)
            return carry

        lax.fori_loop(0, K, body, 0)

    adjt = jnp.where(mask_ref[...] > 0, b, 0.0)
    att = adjt + (rowidx == lax.broadcasted_iota(jnp.int32, (N, N), 1)
                  ).astype(f32)
    sums = jnp.sum(att, axis=0, keepdims=True)
    anormt_ref[...] = _bf(att / sums)


def _main_kernel(x_ref, anormt_ref, w_in_ref, b_in_ref, wa8_ref, wb8_ref,
                 wg8_ref, mlpb8_ref, gen_b_ref, out_ref, planes_ref):
    f32 = jnp.float32
    xb = x_ref[0]
    w_in = _bf(w_in_ref[...].reshape(1, C1, 1)).astype(f32)
    b_in = b_in_ref[...].reshape(1, C1, 1)
    xb16 = _bf(xb).astype(f32)
    h0 = _leaky(xb16[:, None, :] * w_in + b_in)
    h0f = h0.reshape(T * C1, N)
    h0f16 = _bf(h0f)
    prop = _dot16(h0f16, anormt_ref[...], ((1,), (0,)))
    h1f16 = _bf(PROP_ALPHA * h0f + (1.0 - PROP_ALPHA) * prop)

    wa8 = wa8_ref[...]
    wb8 = wb8_ref[...]
    wg8 = wg8_ref[...]
    mlpb8 = mlpb8_ref[...][:, None]

    for g in range(NCHUNK):
        rows = slice(g * TG * C1, (g + 1) * TG * C1)
        lat = (_dot16(wa8, h0f16[rows], ((1,), (0,)))
               + _dot16(wb8, h1f16[rows], ((1,), (0,)))
               + mlpb8)
        pchunk = _dot16(wg8, _bf(lat), ((1,), (0,)))
        for p in range(9):
            planes_ref[p, g * TG:(g + 1) * TG, :] = (
                pchunk[p * TG:(p + 1) * TG])

    planes = planes_ref[...]
    tix = lax.broadcasted_iota(jnp.int32, (T, N), 0)
    nix = lax.broadcasted_iota(jnp.int32, (T, N), 1)
    acc = jnp.zeros((T, N), f32)
    for kh in range(3):
        for kw in range(3):
            di, dj = kh - 1, kw - 1
            shifted = planes[kh * 3 + kw]
            if dj:
                shifted = jnp.roll(shifted, -dj, axis=0)
            if di:
                shifted = jnp.roll(shifted, -di, axis=1)
            valid = ((tix + dj >= 0) & (tix + dj < T)
                     & (nix + di >= 0) & (nix + di < N))
            acc = acc + jnp.where(valid, shifted, 0.0)
    out_ref[0] = _leaky(acc + gen_b_ref[...])


def kernel(x, idx, device, emb1, emb2, lin1_w, lin1_b, lin2_w, lin2_b,
           conv_in_w, conv_in_b, mlp_w, mlp_b, gen_w, gen_b):
    del idx, device
    f32 = jnp.float32

    anormt = pl.pallas_call(
        _graph_kernel,
        out_shape=jax.ShapeDtypeStruct((N, N), jnp.bfloat16),
        scratch_shapes=[pltpu.VMEM((N, N), f32), pltpu.VMEM((N, N), f32)],
    )(emb1, emb2, lin1_w, lin1_b, lin2_w, lin2_b)

    eye8 = jnp.eye(TG, dtype=jnp.bfloat16)
    mw = _bf(mlp_w.reshape(C1, 2 * C1))
    wa8 = jnp.kron(eye8, mw[:, :C1])
    wb8 = jnp.kron(eye8, mw[:, C1:])
    gwt = _bf(gen_w.reshape(C1, 9)).T
    perm = jnp.arange(9 * TG).reshape(9, TG)
    perm = (perm % TG) * 9 + perm // TG
    wg8 = jnp.kron(eye8, gwt)[perm.reshape(-1)]
    mlpb8 = jnp.tile(mlp_b, TG)

    out = pl.pallas_call(
        _main_kernel,
        grid=(x.shape[0],),
        in_specs=[
            pl.BlockSpec((1, T, N), lambda b: (b, 0, 0)),
            pl.BlockSpec((N, N), lambda b: (0, 0)),
            pl.BlockSpec(conv_in_w.shape, lambda b: (0,) * 4),
            pl.BlockSpec(conv_in_b.shape, lambda b: (0,)),
            pl.BlockSpec(wa8.shape, lambda b: (0, 0)),
            pl.BlockSpec(wb8.shape, lambda b: (0, 0)),
            pl.BlockSpec(wg8.shape, lambda b: (0, 0)),
            pl.BlockSpec(mlpb8.shape, lambda b: (0,)),
            pl.BlockSpec(gen_b.shape, lambda b: (0,)),
        ],
        out_specs=pl.BlockSpec((1, T, N), lambda b: (b, 0, 0)),
        out_shape=jax.ShapeDtypeStruct((x.shape[0], T, N), f32),
        scratch_shapes=[pltpu.VMEM((9, T, N), f32)],
    )(x, anormt, conv_in_w, conv_in_b, wa8, wb8, wg8, mlpb8, gen_b)
    return out


# speedup vs baseline: 2.2887x; 1.0372x over previous
import jax
import jax.numpy as jnp
from jax import lax
from jax.experimental import pallas as pl
from jax.experimental.pallas import tpu as pltpu

N = 1024
D = 64
K = 30
ALPHA = 3.0
PROP_ALPHA = 0.05
C1 = 32
T = 48
TG = 8
NCHUNK = T // TG


def _leaky(v):
    return jnp.where(v >= 0, v, 0.01 * v)


def _bf(v):
    return v.astype(jnp.bfloat16)


def _dot16(a16, b16, dims):
    return lax.dot_general(a16, b16, (dims, ((), ())),
                           preferred_element_type=jnp.float32)


def _dot(a, b, dims):
    return _dot16(_bf(a), _bf(b), dims)


def _fused_kernel(x_ref, emb1_ref, emb2_ref, l1w_ref, l1b_ref, l2w_ref,
                  l2b_ref, w_in_ref, b_in_ref, wa8_ref, wb8_ref, wg8_ref,
                  mlpb8_ref, gen_b_ref, out_ref,
                  anormt_ref, work_ref, mask_ref, planes_ref):
    f32 = jnp.float32
    pid = pl.program_id(0)

    @pl.when(pid == 0)
    def _graph():
        n1 = jnp.tanh(ALPHA * (_dot(emb1_ref[...], l1w_ref[...],
                                    ((1,), (1,)))
                               + l1b_ref[...][None, :]))
        n2 = jnp.tanh(ALPHA * (_dot(emb2_ref[...], l2w_ref[...],
                                    ((1,), (1,)))
                               + l2b_ref[...][None, :]))
        s1 = _dot(n1, n2, ((1,), (1,)))
        s2 = _dot(n2, n1, ((1,), (1,)))
        b = jax.nn.relu(jnp.tanh(ALPHA * (s2 - s1)))

        rowidx = lax.broadcasted_iota(jnp.int32, (N, N), 0)

        ones = b == 1.0
        cnt1 = jnp.sum(ones.astype(jnp.int32), axis=0)
        all_saturated = jnp.min(cnt1) >= K

        @pl.when(all_saturated)
        def _fast():
            ltri = (rowidx >= lax.broadcasted_iota(jnp.int32, (N, N), 1)
                    ).astype(jnp.bfloat16)
            csum = _dot16(ltri, ones.astype(jnp.bfloat16), ((1,), (0,)))
            mask_ref[...] = (ones & (csum <= K)).astype(f32)

        @pl.when(jnp.logical_not(all_saturated))
        def _general():
            work_ref[...] = b
            mask_ref[...] = jnp.zeros((N, N), f32)

            def body(_, carry):
                work = work_ref[...]
                m = jnp.max(work, axis=0, keepdims=True)
                is_max = work == m
                sel_row = jnp.min(jnp.where(is_max, rowidx, N), axis=0,
                                  keepdims=True)
                sel = rowidx == sel_row
                mask_ref[...] = mask_ref[...] + sel.astype(f32)
                work_ref[...] = jnp.where(sel, -1.0, work)
                return carry

            lax.fori_loop(0, K, body, 0)

        adjt = jnp.where(mask_ref[...] > 0, b, 0.0)
        att = adjt + (rowidx == lax.broadcasted_iota(jnp.int32, (N, N), 1)
                      ).astype(f32)
        sums = jnp.sum(att, axis=0, keepdims=True)
        anormt_ref[...] = _bf(att / sums)

    @pl.when(pid > 0)
    def _main():
        xb = x_ref[0]
        w_in = _bf(w_in_ref[...].reshape(1, C1, 1)).astype(f32)
        b_in = b_in_ref[...].reshape(1, C1, 1)
        xb16 = _bf(xb).astype(f32)
        h0 = _leaky(xb16[:, None, :] * w_in + b_in)
        h0f = h0.reshape(T * C1, N)
        h0f16 = _bf(h0f)
        prop = _dot16(h0f16, anormt_ref[...], ((1,), (0,)))
        h1f16 = _bf(PROP_ALPHA * h0f + (1.0 - PROP_ALPHA) * prop)

        wa8 = wa8_ref[...]
        wb8 = wb8_ref[...]
        wg8 = wg8_ref[...]
        mlpb8 = mlpb8_ref[...][:, None]

        for g in range(NCHUNK):
            rows = slice(g * TG * C1, (g + 1) * TG * C1)
            lat = (_dot16(wa8, h0f16[rows], ((1,), (0,)))
                   + _dot16(wb8, h1f16[rows], ((1,), (0,)))
                   + mlpb8)
            pchunk = _dot16(wg8, _bf(lat), ((1,), (0,)))
            for p in range(9):
                planes_ref[p, g * TG:(g + 1) * TG, :] = (
                    pchunk[p * TG:(p + 1) * TG])

        planes = planes_ref[...]
        tix = lax.broadcasted_iota(jnp.int32, (T, N), 0)
        nix = lax.broadcasted_iota(jnp.int32, (T, N), 1)
        acc = jnp.zeros((T, N), f32)
        for kh in range(3):
            for kw in range(3):
                di, dj = kh - 1, kw - 1
                shifted = planes[kh * 3 + kw]
                if dj:
                    shifted = jnp.roll(shifted, -dj, axis=0)
                if di:
                    shifted = jnp.roll(shifted, -di, axis=1)
                valid = ((tix + dj >= 0) & (tix + dj < T)
                         & (nix + di >= 0) & (nix + di < N))
                acc = acc + jnp.where(valid, shifted, 0.0)
        out_ref[0] = _leaky(acc + gen_b_ref[...])


def kernel(x, idx, device, emb1, emb2, lin1_w, lin1_b, lin2_w, lin2_b,
           conv_in_w, conv_in_b, mlp_w, mlp_b, gen_w, gen_b):
    del idx, device
    f32 = jnp.float32
    B = x.shape[0]

    eye8 = jnp.eye(TG, dtype=jnp.bfloat16)
    mw = _bf(mlp_w.reshape(C1, 2 * C1))
    wa8 = jnp.kron(eye8, mw[:, :C1])
    wb8 = jnp.kron(eye8, mw[:, C1:])
    gwt = _bf(gen_w.reshape(C1, 9)).T
    perm = jnp.arange(9 * TG).reshape(9, TG)
    perm = (perm % TG) * 9 + perm // TG
    wg8 = jnp.kron(eye8, gwt)[perm.reshape(-1)]
    mlpb8 = jnp.tile(mlp_b, TG)

    bspec = pl.BlockSpec
    out = pl.pallas_call(
        _fused_kernel,
        grid=(1 + B,),
        in_specs=[
            bspec((1, T, N), lambda b: (jnp.maximum(b - 1, 0), 0, 0)),
            bspec(emb1.shape, lambda b: (0, 0)),
            bspec(emb2.shape, lambda b: (0, 0)),
            bspec(lin1_w.shape, lambda b: (0, 0)),
            bspec(lin1_b.shape, lambda b: (0,)),
            bspec(lin2_w.shape, lambda b: (0, 0)),
            bspec(lin2_b.shape, lambda b: (0,)),
            bspec(conv_in_w.shape, lambda b: (0,) * 4),
            bspec(conv_in_b.shape, lambda b: (0,)),
            bspec(wa8.shape, lambda b: (0, 0)),
            bspec(wb8.shape, lambda b: (0, 0)),
            bspec(wg8.shape, lambda b: (0, 0)),
            bspec(mlpb8.shape, lambda b: (0,)),
            bspec(gen_b.shape, lambda b: (0,)),
        ],
        out_specs=bspec((1, T, N), lambda b: (jnp.maximum(b - 1, 0), 0, 0)),
        out_shape=jax.ShapeDtypeStruct((B, T, N), f32),
        scratch_shapes=[
            pltpu.VMEM((N, N), jnp.bfloat16),
            pltpu.VMEM((N, N), f32),
            pltpu.VMEM((N, N), f32),
            pltpu.VMEM((9, T, N), f32),
        ],
    )(x, emb1, emb2, lin1_w, lin1_b, lin2_w, lin2_b, conv_in_w, conv_in_b,
      wa8, wb8, wg8, mlpb8, gen_b)
    return out


# speedup vs baseline: 2.3148x; 1.0114x over previous
import jax
import jax.numpy as jnp
from jax import lax
from jax.experimental import pallas as pl
from jax.experimental.pallas import tpu as pltpu

N = 1024
D = 64
K = 30
ALPHA = 3.0
PROP_ALPHA = 0.05
C1 = 32
T = 48
TG = 8
NCHUNK = T // TG


def _leaky(v):
    return jnp.where(v >= 0, v, 0.01 * v)


def _bf(v):
    return v.astype(jnp.bfloat16)


def _dot16(a16, b16, dims):
    return lax.dot_general(a16, b16, (dims, ((), ())),
                           preferred_element_type=jnp.float32)


def _dot(a, b, dims):
    return _dot16(_bf(a), _bf(b), dims)


def _fused_kernel(x_ref, emb1_ref, emb2_ref, l1w_ref, l1b_ref, l2w_ref,
                  l2b_ref, w_in_ref, b_in_ref, wa8_ref, wb8_ref, wg8_ref,
                  mlpb8_ref, gen_b_ref, out_ref,
                  anormt_ref, work_ref, mask_ref, planes_ref):
    f32 = jnp.float32
    pid = pl.program_id(0)

    @pl.when(pid == 0)
    def _graph():
        n1 = jnp.tanh(ALPHA * (_dot(emb1_ref[...], l1w_ref[...],
                                    ((1,), (1,)))
                               + l1b_ref[...][None, :]))
        n2 = jnp.tanh(ALPHA * (_dot(emb2_ref[...], l2w_ref[...],
                                    ((1,), (1,)))
                               + l2b_ref[...][None, :]))
        s1 = _dot(n1, n2, ((1,), (1,)))
        s2 = _dot(n2, n1, ((1,), (1,)))
        b = jax.nn.relu(jnp.tanh(ALPHA * (s2 - s1)))

        rowidx = lax.broadcasted_iota(jnp.int32, (N, N), 0)

        ones = b == 1.0
        cnt1 = jnp.sum(ones.astype(jnp.int32), axis=0)
        all_saturated = jnp.min(cnt1) >= K

        eye = (rowidx == lax.broadcasted_iota(jnp.int32, (N, N), 1)
               ).astype(f32)

        @pl.when(all_saturated)
        def _fast():
            ltri = (rowidx >= lax.broadcasted_iota(jnp.int32, (N, N), 1)
                    ).astype(jnp.bfloat16)
            csum = _dot16(ltri, ones.astype(jnp.bfloat16), ((1,), (0,)))
            mask = (ones & (csum <= K)).astype(f32)
            anormt_ref[...] = _bf((mask + eye) * (1.0 / (K + 1.0)))

        @pl.when(jnp.logical_not(all_saturated))
        def _general():
            work_ref[...] = b
            mask_ref[...] = jnp.zeros((N, N), f32)

            def body(_, carry):
                work = work_ref[...]
                m = jnp.max(work, axis=0, keepdims=True)
                is_max = work == m
                sel_row = jnp.min(jnp.where(is_max, rowidx, N), axis=0,
                                  keepdims=True)
                sel = rowidx == sel_row
                mask_ref[...] = mask_ref[...] + sel.astype(f32)
                work_ref[...] = jnp.where(sel, -1.0, work)
                return carry

            lax.fori_loop(0, K, body, 0)

            adjt = jnp.where(mask_ref[...] > 0, b, 0.0)
            att = adjt + eye
            sums = jnp.sum(att, axis=0, keepdims=True)
            anormt_ref[...] = _bf(att / sums)

    @pl.when(pid > 0)
    def _main():
        xb = x_ref[0]
        w_in = _bf(w_in_ref[...].reshape(1, C1, 1)).astype(f32)
        b_in = b_in_ref[...].reshape(1, C1, 1)
        xb16 = _bf(xb).astype(f32)
        h0 = _leaky(xb16[:, None, :] * w_in + b_in)
        h0f = h0.reshape(T * C1, N)
        h0f16 = _bf(h0f)
        prop = _dot16(h0f16, anormt_ref[...], ((1,), (0,)))
        h1f16 = _bf(PROP_ALPHA * h0f + (1.0 - PROP_ALPHA) * prop)

        wa8 = wa8_ref[...]
        wb8 = wb8_ref[...]
        wg8 = wg8_ref[...]
        mlpb8 = mlpb8_ref[...][:, None]

        for g in range(NCHUNK):
            rows = slice(g * TG * C1, (g + 1) * TG * C1)
            lat = (_dot16(wa8, h0f16[rows], ((1,), (0,)))
                   + _dot16(wb8, h1f16[rows], ((1,), (0,)))
                   + mlpb8)
            pchunk = _dot16(wg8, _bf(lat), ((1,), (0,)))
            for p in range(9):
                planes_ref[p, g * TG:(g + 1) * TG, :] = (
                    pchunk[p * TG:(p + 1) * TG])

        planes = planes_ref[...]
        tix = lax.broadcasted_iota(jnp.int32, (T, N), 0)
        nix = lax.broadcasted_iota(jnp.int32, (T, N), 1)
        acc = jnp.zeros((T, N), f32)
        for kh in range(3):
            for kw in range(3):
                di, dj = kh - 1, kw - 1
                shifted = planes[kh * 3 + kw]
                if dj:
                    shifted = jnp.roll(shifted, -dj, axis=0)
                if di:
                    shifted = jnp.roll(shifted, -di, axis=1)
                valid = ((tix + dj >= 0) & (tix + dj < T)
                         & (nix + di >= 0) & (nix + di < N))
                acc = acc + jnp.where(valid, shifted, 0.0)
        out_ref[0] = _leaky(acc + gen_b_ref[...])


def kernel(x, idx, device, emb1, emb2, lin1_w, lin1_b, lin2_w, lin2_b,
           conv_in_w, conv_in_b, mlp_w, mlp_b, gen_w, gen_b):
    del idx, device
    f32 = jnp.float32
    B = x.shape[0]

    eye8 = jnp.eye(TG, dtype=jnp.bfloat16)
    mw = _bf(mlp_w.reshape(C1, 2 * C1))
    wa8 = jnp.kron(eye8, mw[:, :C1])
    wb8 = jnp.kron(eye8, mw[:, C1:])
    gwt = _bf(gen_w.reshape(C1, 9)).T
    perm = jnp.arange(9 * TG).reshape(9, TG)
    perm = (perm % TG) * 9 + perm // TG
    wg8 = jnp.kron(eye8, gwt)[perm.reshape(-1)]
    mlpb8 = jnp.tile(mlp_b, TG)

    bspec = pl.BlockSpec
    out = pl.pallas_call(
        _fused_kernel,
        grid=(1 + B,),
        in_specs=[
            bspec((1, T, N), lambda b: (jnp.maximum(b - 1, 0), 0, 0)),
            bspec(emb1.shape, lambda b: (0, 0)),
            bspec(emb2.shape, lambda b: (0, 0)),
            bspec(lin1_w.shape, lambda b: (0, 0)),
            bspec(lin1_b.shape, lambda b: (0,)),
            bspec(lin2_w.shape, lambda b: (0, 0)),
            bspec(lin2_b.shape, lambda b: (0,)),
            bspec(conv_in_w.shape, lambda b: (0,) * 4),
            bspec(conv_in_b.shape, lambda b: (0,)),
            bspec(wa8.shape, lambda b: (0, 0)),
            bspec(wb8.shape, lambda b: (0, 0)),
            bspec(wg8.shape, lambda b: (0, 0)),
            bspec(mlpb8.shape, lambda b: (0,)),
            bspec(gen_b.shape, lambda b: (0,)),
        ],
        out_specs=bspec((1, T, N), lambda b: (jnp.maximum(b - 1, 0), 0, 0)),
        out_shape=jax.ShapeDtypeStruct((B, T, N), f32),
        scratch_shapes=[
            pltpu.VMEM((N, N), jnp.bfloat16),
            pltpu.VMEM((N, N), f32),
            pltpu.VMEM((N, N), f32),
            pltpu.VMEM((9, T, N), f32),
        ],
    )(x, emb1, emb2, lin1_w, lin1_b, lin2_w, lin2_b, conv_in_w, conv_in_b,
      wa8, wb8, wg8, mlpb8, gen_b)
    return out
